# Initial kernel scaffold; baseline (speedup 1.0000x reference)
import jax
import jax.numpy as jnp
from jax.experimental import pallas as pl


def kernel(pcd, imu, velocity, steering, pcd_mask, params):
    raise NotImplementedError("write your pallas kernel here")


# speedup vs baseline: 10.3696x; 10.3696x over previous
import functools

import jax
import jax.numpy as jnp
from jax import lax
from jax.experimental import pallas as pl
from jax.experimental.pallas import tpu as pltpu

N = 2048
C = 128
HEADS = 4
HD = C // HEADS
K = 16
R = 256
NB = N // R
ALPHA = 0.2
NEG = -1e30
BIGF = 3e9


def _gat_body(x_ref, w_ref, al_ref, ar_ref, g_ref, b_ref, o_ref,
              d_ref, nbr_ref):
    x = x_ref[0]
    h = lax.dot_general(x, w_ref[...], (((1,), (1,)), ((), ())),
                        preferred_element_type=jnp.float32)
    crt = lax.dot_general(ar_ref[...], h, (((0,), (1,)), ((), ())),
                          preferred_element_type=jnp.float32)
    xx = x * x
    ones = jnp.ones((1, C), jnp.float32)
    sqt = lax.dot_general(ones, xx, (((1,), (1,)), ((), ())),
                          preferred_element_type=jnp.float32)
    iota = lax.broadcasted_iota(jnp.int32, (R, N), 1).astype(jnp.float32)

    def row_block(rb, _):
        xs = x_ref[0, pl.ds(rb * R, R), :]
        hs = lax.dot_general(xs, w_ref[...], (((1,), (1,)), ((), ())),
                             preferred_element_type=jnp.float32)
        cls = lax.dot_general(hs, al_ref[...], (((1,), (0,)), ((), ())),
                              preferred_element_type=jnp.float32)
        sq_r = jnp.sum(xs * xs, axis=1, keepdims=True)
        d = sq_r + sqt - 2.0 * lax.dot_general(
            xs, x, (((1,), (1,)), ((), ())),
            preferred_element_type=jnp.float32)
        d_ref[...] = jnp.maximum(d, 0.0)
        nbr_ref[...] = jnp.zeros((R, N), jnp.float32)

        def sel_step(t, _):
            dc = d_ref[...]
            m = jnp.min(dc, axis=1, keepdims=True)
            eq = dc == m
            idxf = jnp.min(jnp.where(eq, iota, BIGF), axis=1, keepdims=True)
            onehot = iota == idxf
            d_ref[...] = jnp.where(onehot, jnp.float32(3.4e38), dc)
            keep = jnp.logical_and(onehot, t > 0)
            nbr_ref[...] = jnp.where(keep, 1.0, nbr_ref[...])
            return 0

        lax.fori_loop(0, K + 1, sel_step, 0)
        nbr = nbr_ref[...] > 0.5

        outs = []
        for hd in range(HEADS):
            e = cls[:, hd:hd + 1] + crt[hd:hd + 1, :]
            e = jnp.where(e > 0, e, ALPHA * e)
            e = jnp.where(nbr, e, NEG)
            mx = jnp.max(e, axis=1, keepdims=True)
            p = jnp.where(nbr, jnp.exp(e - mx), 0.0)
            s = p / jnp.sum(p, axis=1, keepdims=True)
            outs.append(lax.dot_general(
                s, h[:, hd * HD:(hd + 1) * HD], (((1,), (0,)), ((), ())),
                preferred_element_type=jnp.float32))
        attn = jnp.concatenate(outs, axis=1)

        y = attn + xs
        mu = jnp.mean(y, axis=1, keepdims=True)
        var = jnp.mean((y - mu) ** 2, axis=1, keepdims=True)
        yn = (y - mu) / jnp.sqrt(var + 1e-5) * g_ref[...] + b_ref[...]
        o_ref[0, pl.ds(rb * R, R), :] = jnp.maximum(yn, 0.0)
        return 0

    lax.fori_loop(0, NB, row_block, 0)


def _gat_layer(x, W, a, g, b, interpret=False):
    B = x.shape[0]
    aL = a[0, :HD]
    aR = a[0, HD:]
    eye = jnp.eye(HEADS, dtype=jnp.float32)
    aLf = (eye[:, None, :] * aL[None, :, None]).reshape(C, HEADS)
    aRf = (eye[:, None, :] * aR[None, :, None]).reshape(C, HEADS)
    return pl.pallas_call(
        _gat_body,
        grid=(B,),
        in_specs=[
            pl.BlockSpec((1, N, C), lambda i: (i, 0, 0)),
            pl.BlockSpec((C, C), lambda i: (0, 0)),
            pl.BlockSpec((C, HEADS), lambda i: (0, 0)),
            pl.BlockSpec((C, HEADS), lambda i: (0, 0)),
            pl.BlockSpec((1, C), lambda i: (0, 0)),
            pl.BlockSpec((1, C), lambda i: (0, 0)),
        ],
        out_specs=pl.BlockSpec((1, N, C), lambda i: (i, 0, 0)),
        out_shape=jax.ShapeDtypeStruct((B, N, C), jnp.float32),
        scratch_shapes=[pltpu.VMEM((R, N), jnp.float32),
                        pltpu.VMEM((R, N), jnp.float32)],
        interpret=interpret,
    )(x, W, aLf, aRf, g.reshape(1, C), b.reshape(1, C))


def _gru(x, Wih, Whh, bih, bhh):
    H = Whh.shape[1]
    B = x.shape[0]

    def step(h, xt):
        gi = xt @ Wih.T + bih
        gh = h @ Whh.T + bhh
        i_r, i_z, i_n = jnp.split(gi, 3, axis=-1)
        h_r, h_z, h_n = jnp.split(gh, 3, axis=-1)
        r = jax.nn.sigmoid(i_r + h_r)
        z = jax.nn.sigmoid(i_z + h_z)
        n = jnp.tanh(i_n + r * h_n)
        return (1.0 - z) * n + z * h, None

    h0 = jnp.zeros((B, H), x.dtype)
    hT, _ = jax.lax.scan(step, h0, jnp.swapaxes(x, 0, 1))
    return hT


def _layernorm(x, g, b, eps=1e-5):
    m = jnp.mean(x, axis=-1, keepdims=True)
    v = jnp.mean((x - m) ** 2, axis=-1, keepdims=True)
    return (x - m) / jnp.sqrt(v + eps) * g + b


def _mha(q, kv, in_W, in_b, out_W, out_b, heads=4):
    E = q.shape[-1]
    Wq, Wk, Wv = jnp.split(in_W, 3, axis=0)
    bq, bk, bv = jnp.split(in_b, 3, axis=0)
    B, Lq, _ = q.shape
    Lk = kv.shape[1]
    hd = E // heads
    Q = (q @ Wq.T + bq).reshape(B, Lq, heads, hd).transpose(0, 2, 1, 3)
    Kk = (kv @ Wk.T + bk).reshape(B, Lk, heads, hd).transpose(0, 2, 1, 3)
    V = (kv @ Wv.T + bv).reshape(B, Lk, heads, hd).transpose(0, 2, 1, 3)
    scores = (Q @ jnp.swapaxes(Kk, -1, -2)) / jnp.sqrt(jnp.float32(hd))
    A = jax.nn.softmax(scores, axis=-1)
    O = (A @ V).transpose(0, 2, 1, 3).reshape(B, Lq, E)
    return O @ out_W.T + out_b


def kernel(pcd, imu, velocity, steering, pcd_mask, params):
    p = params
    temporal_in = jnp.concatenate([imu, velocity, steering], axis=-1)
    B, T, M, Ci = temporal_in.shape
    frame = _gru(temporal_in.reshape(B * T, M, Ci), p['micro_Wih'],
                 p['micro_Whh'], p['micro_bih'], p['micro_bhh'])
    frame = frame.reshape(B, T, -1)
    temporal_feat = _gru(frame, p['macro_Wih'], p['macro_Whh'],
                         p['macro_bih'], p['macro_bhh'])
    x = jax.nn.relu(pcd @ p['mlp_W1'].T + p['mlp_b1'])
    x = jax.nn.relu(x @ p['mlp_W2'].T + p['mlp_b2'])
    for i in range(3):
        x = _gat_layer(x, p['gat_W%d' % i], p['gat_a%d' % i],
                       p['ln_g%d' % i], p['ln_b%d' % i])
    spat_tokens = x
    tproj = temporal_feat @ p['tproj_W'].T + p['tproj_b']
    Q = tproj[:, None, :]
    Kt = spat_tokens @ p['sproj_W'].T + p['sproj_b']
    attn_out = _mha(Q, Kt, p['mha_in_W'], p['mha_in_b'],
                    p['mha_out_W'], p['mha_out_b'])
    fused = _layernorm(attn_out[:, 0, :] + tproj, p['fnorm_g'], p['fnorm_b'])
    h = jax.nn.relu(fused @ p['head_W1'].T + p['head_b1'])
    pred = h @ p['head_W2'].T + p['head_b2']
    return pred
